# baseline (speedup 1.0000x reference)
import functools

import jax
import jax.numpy as jnp
from jax import lax
from jax.experimental import pallas as pl
from jax.experimental.pallas import tpu as pltpu
from jax.experimental.pallas import tpu_sc as plsc

N = 10000
E = 320000
D = 128
H = 64
O = 2
W1 = H + 16

NC = 2
NS = 16
NW = NC * NS

LANES = 128
KJ = 4
KH = KJ // 2
R = 10240
ROWS_PER_TILE = R // NS
E_PAD = 327680
IDX_ROWS = E_PAD // LANES
ROWS_PER_WORKER = IDX_ROWS // NW
STEPS = ROWS_PER_WORKER // KJ

BN = 1000
GRID = N // BN


def _make_sc_aggregate(width):
    mesh = plsc.VectorSubcoreMesh(core_axis_name="c", subcore_axis_name="s",
                                  num_cores=NC, num_subcores=NS)

    def body(p_hbm, src_hbm, dst_hbm, agg_out,
             src_v, dst_v, rows_v, acc_sh, sem_g, sem_s, sem_i):
        cid = lax.axis_index("c")
        sid = lax.axis_index("s")
        wid = cid * NS + sid
        r0 = sid * ROWS_PER_TILE
        base = wid * ROWS_PER_WORKER

        ci0 = pltpu.async_copy(src_hbm.at[pl.ds(base, ROWS_PER_WORKER)],
                               src_v, sem_i)
        ci1 = pltpu.async_copy(dst_hbm.at[pl.ds(base, ROWS_PER_WORKER)],
                               dst_v, sem_i)

        def zstore(i, carry):
            for k in range(width // 16):
                rows_v[0, i, pl.ds(k * 16, 16)] = jnp.zeros((16,),
                                                            jnp.float32)
            return carry

        lax.fori_loop(0, LANES, zstore, 0)
        for q in range(ROWS_PER_TILE // LANES):
            pltpu.sync_copy(rows_v.at[0],
                            acc_sh.at[pl.ds(r0 + q * LANES, LANES)])
        ci0.wait()
        ci1.wait()
        plsc.subcore_barrier()

        def step(t, carry):
            row = t * KJ
            g0 = [pltpu.async_copy(p_hbm.at[src_v.at[row + j]],
                                   rows_v.at[j], sem_g)
                  for j in range(KH)]
            for h in g0:
                h.wait()
            s0 = [pltpu.async_copy(rows_v.at[j],
                                   acc_sh.at[dst_v.at[row + j]],
                                   sem_s, add=True)
                  for j in range(KH)]
            g1 = [pltpu.async_copy(p_hbm.at[src_v.at[row + KH + j]],
                                   rows_v.at[KH + j], sem_g)
                  for j in range(KH)]
            for h in g1:
                h.wait()
            s1 = [pltpu.async_copy(rows_v.at[KH + j],
                                   acc_sh.at[dst_v.at[row + KH + j]],
                                   sem_s, add=True)
                  for j in range(KH)]
            for h in s0 + s1:
                h.wait()
            return carry

        lax.fori_loop(0, STEPS, step, 0)
        plsc.subcore_barrier()

        pltpu.sync_copy(acc_sh.at[pl.ds(r0, ROWS_PER_TILE)],
                        agg_out.at[cid, pl.ds(r0, ROWS_PER_TILE)])

    return pl.kernel(
        body,
        out_type=jax.ShapeDtypeStruct((NC, R, width), jnp.float32),
        mesh=mesh,
        scratch_types=(
            pltpu.VMEM((ROWS_PER_WORKER, LANES), jnp.int32),
            pltpu.VMEM((ROWS_PER_WORKER, LANES), jnp.int32),
            pltpu.VMEM((KJ, LANES, width), jnp.float32),
            pltpu.VMEM_SHARED((R, width), jnp.float32),
            pltpu.SemaphoreType.DMA,
            pltpu.SemaphoreType.DMA,
            pltpu.SemaphoreType.DMA,
        ),
        compiler_params=pltpu.CompilerParams(use_tc_tiling_on_sc=False))


_sc_agg_w1 = _make_sc_aggregate(W1)
_sc_agg_w2 = _make_sc_aggregate(H)


def _proj1_body(x_ref, wa_ref, wb_ref, oa_ref, ob_ref):
    xb = x_ref[...]
    oa_ref[...] = jnp.dot(xb, wa_ref[...], preferred_element_type=jnp.float32)
    pb = jnp.dot(xb, wb_ref[...], preferred_element_type=jnp.float32)
    ob_ref[...] = jnp.concatenate(
        [pb, jnp.ones((BN, W1 - H), jnp.float32)], axis=1)


def _proj1(x, wa, wb):
    return pl.pallas_call(
        _proj1_body,
        grid=(GRID,),
        in_specs=[
            pl.BlockSpec((BN, D), lambda i: (i, 0)),
            pl.BlockSpec((D, H), lambda i: (0, 0)),
            pl.BlockSpec((D, H), lambda i: (0, 0)),
        ],
        out_specs=[
            pl.BlockSpec((BN, H), lambda i: (i, 0)),
            pl.BlockSpec((BN, W1), lambda i: (i, 0)),
        ],
        out_shape=[
            jax.ShapeDtypeStruct((N, H), jnp.float32),
            jax.ShapeDtypeStruct((N, W1), jnp.float32),
        ],
    )(x, wa, wb)


def _layer2_body(s_ref, aggp_ref, b_ref, wa_ref, wb_ref,
                 oa_ref, ob_ref, od_ref):
    comb = aggp_ref[0] + aggp_ref[1]
    deg = comb[:, H:H + 1]
    h = jnp.maximum(
        s_ref[...] + comb[:, :H] / jnp.maximum(deg, 1.0) + b_ref[...], 0.0)
    oa_ref[...] = jnp.dot(h, wa_ref[...], preferred_element_type=jnp.float32)
    ob_ref[...] = jnp.dot(h, wb_ref[...], preferred_element_type=jnp.float32)
    od_ref[...] = comb[:, H:]


def _layer2(s, aggp, b, wa, wb):
    return pl.pallas_call(
        _layer2_body,
        grid=(GRID,),
        in_specs=[
            pl.BlockSpec((BN, H), lambda i: (i, 0)),
            pl.BlockSpec((NC, BN, W1), lambda i: (0, i, 0)),
            pl.BlockSpec((1, H), lambda i: (0, 0)),
            pl.BlockSpec((H, H), lambda i: (0, 0)),
            pl.BlockSpec((H, H), lambda i: (0, 0)),
        ],
        out_specs=[
            pl.BlockSpec((BN, H), lambda i: (i, 0)),
            pl.BlockSpec((BN, H), lambda i: (i, 0)),
            pl.BlockSpec((BN, W1 - H), lambda i: (i, 0)),
        ],
        out_shape=[
            jax.ShapeDtypeStruct((N, H), jnp.float32),
            jax.ShapeDtypeStruct((N, H), jnp.float32),
            jax.ShapeDtypeStruct((N, W1 - H), jnp.float32),
        ],
    )(s, aggp, b, wa, wb)


def _head_body(s_ref, aggp_ref, deg_ref, b_ref, wh_ref, bh_ref, o_ref):
    agg = aggp_ref[0] + aggp_ref[1]
    deg = deg_ref[:, 0:1]
    h = jnp.maximum(
        s_ref[...] + agg / jnp.maximum(deg, 1.0) + b_ref[...], 0.0)
    o_ref[...] = (jnp.dot(h, wh_ref[...], preferred_element_type=jnp.float32)
                  + bh_ref[...])


def _head(s, aggp, deg, b, wh_pad, bh_pad):
    return pl.pallas_call(
        _head_body,
        grid=(GRID,),
        in_specs=[
            pl.BlockSpec((BN, H), lambda i: (i, 0)),
            pl.BlockSpec((NC, BN, H), lambda i: (0, i, 0)),
            pl.BlockSpec((BN, W1 - H), lambda i: (i, 0)),
            pl.BlockSpec((1, H), lambda i: (0, 0)),
            pl.BlockSpec((H, 128), lambda i: (0, 0)),
            pl.BlockSpec((1, 128), lambda i: (0, 0)),
        ],
        out_specs=pl.BlockSpec((BN, 128), lambda i: (i, 0)),
        out_shape=jax.ShapeDtypeStruct((N, 128), jnp.float32),
    )(s, aggp, deg, b, wh_pad, bh_pad)


def kernel(x, edge_index, W_self1, W_neigh1, b1, W_self2, W_neigh2, b2,
           W_head, b_head):
    src = edge_index[0].astype(jnp.int32)
    dst = edge_index[1].astype(jnp.int32)
    pad = E_PAD - E
    src_p = jnp.concatenate([src, jnp.zeros((pad,), jnp.int32)])
    pad_dst = N + (jnp.arange(pad, dtype=jnp.int32) % (R - N))
    dst_p = jnp.concatenate([dst, pad_dst])
    src_p = src_p.reshape(IDX_ROWS, LANES)
    dst_p = dst_p.reshape(IDX_ROWS, LANES)

    s1, p1 = _proj1(x, W_self1, W_neigh1)
    aggp1 = _sc_agg_w1(p1, src_p, dst_p)

    s2, p2, deg = _layer2(s1, aggp1, b1.reshape(1, H), W_self2, W_neigh2)

    aggp2 = _sc_agg_w2(p2, src_p, dst_p)

    wh_pad = jnp.zeros((H, 128), jnp.float32).at[:, :O].set(W_head)
    bh_pad = jnp.zeros((1, 128), jnp.float32).at[:, :O].set(
        b_head.reshape(1, O))
    out_pad = _head(s2, aggp2, deg, b2.reshape(1, H), wh_pad, bh_pad)
    return out_pad[:, :O]


# speedup vs baseline: 2.8355x; 2.8355x over previous
import functools

import jax
import jax.numpy as jnp
from jax import lax
from jax.experimental import pallas as pl
from jax.experimental.pallas import tpu as pltpu
from jax.experimental.pallas import tpu_sc as plsc

N = 10000
E = 320000
D = 128
H = 64
O = 2
W1 = H + 16

NC = 2
NS = 16
NW = NC * NS

LANES = 128
KJ = 2
KH = KJ // 2
CH = 20
R = 10240
ROWS_PER_TILE = R // NS
E_PAD = 327680
IDX_ROWS = E_PAD // LANES
ROWS_PER_WORKER = IDX_ROWS // NW
STEPS = ROWS_PER_WORKER // KJ

BN = 1000
GRID = N // BN


def _make_sc_aggregate(width):
    mesh = plsc.VectorSubcoreMesh(core_axis_name="c", subcore_axis_name="s",
                                  num_cores=NC, num_subcores=NS)
    t_rows = N // NS

    def body(p_hbm, src_hbm, dst_hbm, agg_out,
             srcc_v, dstc_v, rows_v, table_sh, acc_sh, sem_g, sem_s, sem_i):
        cid = lax.axis_index("c")
        sid = lax.axis_index("s")
        wid = cid * NS + sid
        r0 = sid * ROWS_PER_TILE
        base = wid * ROWS_PER_WORKER

        ct = pltpu.async_copy(p_hbm.at[pl.ds(sid * t_rows, t_rows)],
                              table_sh.at[pl.ds(sid * t_rows, t_rows)],
                              sem_i)

        def zstore(i, carry):
            for k in range(width // 16):
                rows_v[0, pl.ds(i * 2, 2), pl.ds(k * 16, 16)] = jnp.zeros(
                    (2, 16), jnp.bfloat16)
            return carry

        lax.fori_loop(0, LANES // 2, zstore, 0)
        for q in range(ROWS_PER_TILE // LANES):
            pltpu.sync_copy(rows_v.at[0],
                            acc_sh.at[pl.ds(r0 + q * LANES, LANES)])
        ct.wait()
        plsc.subcore_barrier()

        def chunk(c, carry):
            crow = base + c * CH
            ci0 = pltpu.async_copy(src_hbm.at[pl.ds(crow, CH)], srcc_v,
                                   sem_i)
            ci1 = pltpu.async_copy(dst_hbm.at[pl.ds(crow, CH)], dstc_v,
                                   sem_i)
            ci0.wait()
            ci1.wait()

            def sub(u, carry2):
                row = u * KJ
                g0 = [pltpu.async_copy(table_sh.at[srcc_v.at[row + j]],
                                       rows_v.at[j], sem_g)
                      for j in range(KH)]
                for h in g0:
                    h.wait()
                s0 = [pltpu.async_copy(rows_v.at[j],
                                       acc_sh.at[dstc_v.at[row + j]],
                                       sem_s, add=True)
                      for j in range(KH)]
                g1 = [pltpu.async_copy(table_sh.at[srcc_v.at[row + KH + j]],
                                       rows_v.at[KH + j], sem_g)
                      for j in range(KH)]
                for h in g1:
                    h.wait()
                s1 = [pltpu.async_copy(rows_v.at[KH + j],
                                       acc_sh.at[dstc_v.at[row + KH + j]],
                                       sem_s, add=True)
                      for j in range(KH)]
                for h in s0 + s1:
                    h.wait()
                return carry2

            lax.fori_loop(0, CH // KJ, sub, 0)
            return carry

        lax.fori_loop(0, ROWS_PER_WORKER // CH, chunk, 0)
        plsc.subcore_barrier()

        pltpu.sync_copy(acc_sh.at[pl.ds(r0, ROWS_PER_TILE)],
                        agg_out.at[cid, pl.ds(r0, ROWS_PER_TILE)])

    return pl.kernel(
        body,
        out_type=jax.ShapeDtypeStruct((NC, R, width), jnp.bfloat16),
        mesh=mesh,
        scratch_types=(
            pltpu.VMEM((CH, LANES), jnp.int32),
            pltpu.VMEM((CH, LANES), jnp.int32),
            pltpu.VMEM((KJ, LANES, width), jnp.bfloat16),
            pltpu.VMEM_SHARED((N, width), jnp.bfloat16),
            pltpu.VMEM_SHARED((R, width), jnp.bfloat16),
            pltpu.SemaphoreType.DMA,
            pltpu.SemaphoreType.DMA,
            pltpu.SemaphoreType.DMA,
        ),
        compiler_params=pltpu.CompilerParams(use_tc_tiling_on_sc=False))


_sc_agg_w1 = _make_sc_aggregate(W1)
_sc_agg_w2 = _make_sc_aggregate(H)


def _proj1_body(x_ref, wa_ref, wb_ref, oa_ref, ob_ref):
    xb = x_ref[...]
    oa_ref[...] = jnp.dot(xb, wa_ref[...], preferred_element_type=jnp.float32)
    pb = jnp.dot(xb, wb_ref[...], preferred_element_type=jnp.float32)
    ob_ref[...] = jnp.concatenate(
        [pb, jnp.ones((BN, W1 - H), jnp.float32)], axis=1).astype(jnp.bfloat16)


def _proj1(x, wa, wb):
    return pl.pallas_call(
        _proj1_body,
        grid=(GRID,),
        in_specs=[
            pl.BlockSpec((BN, D), lambda i: (i, 0)),
            pl.BlockSpec((D, H), lambda i: (0, 0)),
            pl.BlockSpec((D, H), lambda i: (0, 0)),
        ],
        out_specs=[
            pl.BlockSpec((BN, H), lambda i: (i, 0)),
            pl.BlockSpec((BN, W1), lambda i: (i, 0)),
        ],
        out_shape=[
            jax.ShapeDtypeStruct((N, H), jnp.float32),
            jax.ShapeDtypeStruct((N, W1), jnp.bfloat16),
        ],
    )(x, wa, wb)


def _layer2_body(s_ref, aggp_ref, b_ref, wa_ref, wb_ref,
                 oa_ref, ob_ref, od_ref):
    comb = (aggp_ref[0].astype(jnp.float32)
            + aggp_ref[1].astype(jnp.float32))
    deg = comb[:, H:H + 1]
    h = jnp.maximum(
        s_ref[...] + comb[:, :H] / jnp.maximum(deg, 1.0) + b_ref[...], 0.0)
    oa_ref[...] = jnp.dot(h, wa_ref[...], preferred_element_type=jnp.float32)
    ob_ref[...] = jnp.dot(h, wb_ref[...],
                          preferred_element_type=jnp.float32
                          ).astype(jnp.bfloat16)
    od_ref[...] = comb[:, H:]


def _layer2(s, aggp, b, wa, wb):
    return pl.pallas_call(
        _layer2_body,
        grid=(GRID,),
        in_specs=[
            pl.BlockSpec((BN, H), lambda i: (i, 0)),
            pl.BlockSpec((NC, BN, W1), lambda i: (0, i, 0)),
            pl.BlockSpec((1, H), lambda i: (0, 0)),
            pl.BlockSpec((H, H), lambda i: (0, 0)),
            pl.BlockSpec((H, H), lambda i: (0, 0)),
        ],
        out_specs=[
            pl.BlockSpec((BN, H), lambda i: (i, 0)),
            pl.BlockSpec((BN, H), lambda i: (i, 0)),
            pl.BlockSpec((BN, W1 - H), lambda i: (i, 0)),
        ],
        out_shape=[
            jax.ShapeDtypeStruct((N, H), jnp.float32),
            jax.ShapeDtypeStruct((N, H), jnp.bfloat16),
            jax.ShapeDtypeStruct((N, W1 - H), jnp.float32),
        ],
    )(s, aggp, b, wa, wb)


def _head_body(s_ref, aggp_ref, deg_ref, b_ref, wh_ref, bh_ref, o_ref):
    agg = (aggp_ref[0].astype(jnp.float32)
           + aggp_ref[1].astype(jnp.float32))
    deg = deg_ref[:, 0:1]
    h = jnp.maximum(
        s_ref[...] + agg / jnp.maximum(deg, 1.0) + b_ref[...], 0.0)
    o_ref[...] = (jnp.dot(h, wh_ref[...], preferred_element_type=jnp.float32)
                  + bh_ref[...])


def _head(s, aggp, deg, b, wh, bh):
    return pl.pallas_call(
        _head_body,
        grid=(GRID,),
        in_specs=[
            pl.BlockSpec((BN, H), lambda i: (i, 0)),
            pl.BlockSpec((NC, BN, H), lambda i: (0, i, 0)),
            pl.BlockSpec((BN, W1 - H), lambda i: (i, 0)),
            pl.BlockSpec((1, H), lambda i: (0, 0)),
            pl.BlockSpec((H, O), lambda i: (0, 0)),
            pl.BlockSpec((1, O), lambda i: (0, 0)),
        ],
        out_specs=pl.BlockSpec((BN, O), lambda i: (i, 0)),
        out_shape=jax.ShapeDtypeStruct((N, O), jnp.float32),
    )(s, aggp, deg, b, wh, bh)


def kernel(x, edge_index, W_self1, W_neigh1, b1, W_self2, W_neigh2, b2,
           W_head, b_head):
    src = edge_index[0].astype(jnp.int32)
    dst = edge_index[1].astype(jnp.int32)
    pad = E_PAD - E
    src_p = jnp.concatenate([src, jnp.zeros((pad,), jnp.int32)])
    pad_dst = N + (jnp.arange(pad, dtype=jnp.int32) % (R - N))
    dst_p = jnp.concatenate([dst, pad_dst])
    src_p = src_p.reshape(IDX_ROWS, LANES)
    dst_p = dst_p.reshape(IDX_ROWS, LANES)

    s1, p1 = _proj1(x, W_self1, W_neigh1)
    aggp1 = _sc_agg_w1(p1, src_p, dst_p)

    s2, p2, deg = _layer2(s1, aggp1, b1.reshape(1, H), W_self2, W_neigh2)

    aggp2 = _sc_agg_w2(p2, src_p, dst_p)

    return _head(s2, aggp2, deg, b2.reshape(1, H), W_head,
                 b_head.reshape(1, O))


# speedup vs baseline: 2.9990x; 1.0577x over previous
import functools

import jax
import jax.numpy as jnp
from jax import lax
from jax.experimental import pallas as pl
from jax.experimental.pallas import tpu as pltpu
from jax.experimental.pallas import tpu_sc as plsc

N = 10000
E = 320000
D = 128
H = 64
O = 2
W1 = H + 16

NC = 2
NS = 16
NW = NC * NS

LANES = 128
CH = 10
HB = CH // 2
R = 10240
ROWS_PER_TILE = R // NS
E_PAD = 327680
IDX_ROWS = E_PAD // LANES
ROWS_PER_WORKER = IDX_ROWS // NW

BN = 1000
GRID = N // BN


def _make_sc_aggregate(width):
    mesh = plsc.VectorSubcoreMesh(core_axis_name="c", subcore_axis_name="s",
                                  num_cores=NC, num_subcores=NS)
    t_rows = N // NS

    def body(p_hbm, src_hbm, dst_hbm, agg_out,
             srcc_v, dstc_v, rows_v, table_sh, acc_sh, sem_g, sem_s, sem_i):
        cid = lax.axis_index("c")
        sid = lax.axis_index("s")
        wid = cid * NS + sid
        r0 = sid * ROWS_PER_TILE
        base = wid * ROWS_PER_WORKER

        ct = pltpu.async_copy(p_hbm.at[pl.ds(sid * t_rows, t_rows)],
                              table_sh.at[pl.ds(sid * t_rows, t_rows)],
                              sem_i)

        def zstore(i, carry):
            for k in range(width // 16):
                rows_v[0, pl.ds(i * 2, 2), pl.ds(k * 16, 16)] = jnp.zeros(
                    (2, 16), jnp.bfloat16)
            return carry

        lax.fori_loop(0, LANES // 2, zstore, 0)
        for q in range(ROWS_PER_TILE // LANES):
            pltpu.sync_copy(rows_v.at[0],
                            acc_sh.at[pl.ds(r0 + q * LANES, LANES)])
        ct.wait()
        plsc.subcore_barrier()

        def chunk(c, carry):
            crow = base + c * CH
            ci0 = pltpu.async_copy(src_hbm.at[pl.ds(crow, CH)], srcc_v,
                                   sem_i)
            ci1 = pltpu.async_copy(dst_hbm.at[pl.ds(crow, CH)], dstc_v,
                                   sem_i)
            ci0.wait()
            ci1.wait()

            g0 = [pltpu.async_copy(table_sh.at[srcc_v.at[j]],
                                   rows_v.at[j], sem_g)
                  for j in range(HB)]
            scatters = []
            g1 = []
            for j in range(HB):
                g0[j].wait()
                scatters.append(
                    pltpu.async_copy(rows_v.at[j],
                                     acc_sh.at[dstc_v.at[j]],
                                     sem_s, add=True))
                g1.append(
                    pltpu.async_copy(table_sh.at[srcc_v.at[HB + j]],
                                     rows_v.at[HB + j], sem_g))
            for j in range(HB):
                g1[j].wait()
                scatters.append(
                    pltpu.async_copy(rows_v.at[HB + j],
                                     acc_sh.at[dstc_v.at[HB + j]],
                                     sem_s, add=True))
            for h in scatters:
                h.wait()
            return carry

        lax.fori_loop(0, ROWS_PER_WORKER // CH, chunk, 0)
        plsc.subcore_barrier()

        pltpu.sync_copy(acc_sh.at[pl.ds(r0, ROWS_PER_TILE)],
                        agg_out.at[cid, pl.ds(r0, ROWS_PER_TILE)])

    return pl.kernel(
        body,
        out_type=jax.ShapeDtypeStruct((NC, R, width), jnp.bfloat16),
        mesh=mesh,
        scratch_types=(
            pltpu.VMEM((CH, LANES), jnp.int32),
            pltpu.VMEM((CH, LANES), jnp.int32),
            pltpu.VMEM((CH, LANES, width), jnp.bfloat16),
            pltpu.VMEM_SHARED((N, width), jnp.bfloat16),
            pltpu.VMEM_SHARED((R, width), jnp.bfloat16),
            pltpu.SemaphoreType.DMA,
            pltpu.SemaphoreType.DMA,
            pltpu.SemaphoreType.DMA,
        ),
        compiler_params=pltpu.CompilerParams(use_tc_tiling_on_sc=False))


_sc_agg_w1 = _make_sc_aggregate(W1)
_sc_agg_w2 = _make_sc_aggregate(H)


def _proj1_body(x_ref, wa_ref, wb_ref, oa_ref, ob_ref):
    xb = x_ref[...]
    oa_ref[...] = jnp.dot(xb, wa_ref[...], preferred_element_type=jnp.float32)
    pb = jnp.dot(xb, wb_ref[...], preferred_element_type=jnp.float32)
    ob_ref[...] = jnp.concatenate(
        [pb, jnp.ones((BN, W1 - H), jnp.float32)], axis=1).astype(jnp.bfloat16)


def _proj1(x, wa, wb):
    return pl.pallas_call(
        _proj1_body,
        grid=(GRID,),
        in_specs=[
            pl.BlockSpec((BN, D), lambda i: (i, 0)),
            pl.BlockSpec((D, H), lambda i: (0, 0)),
            pl.BlockSpec((D, H), lambda i: (0, 0)),
        ],
        out_specs=[
            pl.BlockSpec((BN, H), lambda i: (i, 0)),
            pl.BlockSpec((BN, W1), lambda i: (i, 0)),
        ],
        out_shape=[
            jax.ShapeDtypeStruct((N, H), jnp.float32),
            jax.ShapeDtypeStruct((N, W1), jnp.bfloat16),
        ],
    )(x, wa, wb)


def _layer2_body(s_ref, aggp_ref, b_ref, wa_ref, wb_ref,
                 oa_ref, ob_ref, od_ref):
    comb = (aggp_ref[0].astype(jnp.float32)
            + aggp_ref[1].astype(jnp.float32))
    deg = comb[:, H:H + 1]
    h = jnp.maximum(
        s_ref[...] + comb[:, :H] / jnp.maximum(deg, 1.0) + b_ref[...], 0.0)
    oa_ref[...] = jnp.dot(h, wa_ref[...], preferred_element_type=jnp.float32)
    ob_ref[...] = jnp.dot(h, wb_ref[...],
                          preferred_element_type=jnp.float32
                          ).astype(jnp.bfloat16)
    od_ref[...] = comb[:, H:]


def _layer2(s, aggp, b, wa, wb):
    return pl.pallas_call(
        _layer2_body,
        grid=(GRID,),
        in_specs=[
            pl.BlockSpec((BN, H), lambda i: (i, 0)),
            pl.BlockSpec((NC, BN, W1), lambda i: (0, i, 0)),
            pl.BlockSpec((1, H), lambda i: (0, 0)),
            pl.BlockSpec((H, H), lambda i: (0, 0)),
            pl.BlockSpec((H, H), lambda i: (0, 0)),
        ],
        out_specs=[
            pl.BlockSpec((BN, H), lambda i: (i, 0)),
            pl.BlockSpec((BN, H), lambda i: (i, 0)),
            pl.BlockSpec((BN, W1 - H), lambda i: (i, 0)),
        ],
        out_shape=[
            jax.ShapeDtypeStruct((N, H), jnp.float32),
            jax.ShapeDtypeStruct((N, H), jnp.bfloat16),
            jax.ShapeDtypeStruct((N, W1 - H), jnp.float32),
        ],
    )(s, aggp, b, wa, wb)


def _head_body(s_ref, aggp_ref, deg_ref, b_ref, wh_ref, bh_ref, o_ref):
    agg = (aggp_ref[0].astype(jnp.float32)
           + aggp_ref[1].astype(jnp.float32))
    deg = deg_ref[:, 0:1]
    h = jnp.maximum(
        s_ref[...] + agg / jnp.maximum(deg, 1.0) + b_ref[...], 0.0)
    o_ref[...] = (jnp.dot(h, wh_ref[...], preferred_element_type=jnp.float32)
                  + bh_ref[...])


def _head(s, aggp, deg, b, wh, bh):
    return pl.pallas_call(
        _head_body,
        grid=(GRID,),
        in_specs=[
            pl.BlockSpec((BN, H), lambda i: (i, 0)),
            pl.BlockSpec((NC, BN, H), lambda i: (0, i, 0)),
            pl.BlockSpec((BN, W1 - H), lambda i: (i, 0)),
            pl.BlockSpec((1, H), lambda i: (0, 0)),
            pl.BlockSpec((H, O), lambda i: (0, 0)),
            pl.BlockSpec((1, O), lambda i: (0, 0)),
        ],
        out_specs=pl.BlockSpec((BN, O), lambda i: (i, 0)),
        out_shape=jax.ShapeDtypeStruct((N, O), jnp.float32),
    )(s, aggp, deg, b, wh, bh)


def kernel(x, edge_index, W_self1, W_neigh1, b1, W_self2, W_neigh2, b2,
           W_head, b_head):
    src = edge_index[0].astype(jnp.int32)
    dst = edge_index[1].astype(jnp.int32)
    pad = E_PAD - E
    src_p = jnp.concatenate([src, jnp.zeros((pad,), jnp.int32)])
    pad_dst = N + (jnp.arange(pad, dtype=jnp.int32) % (R - N))
    dst_p = jnp.concatenate([dst, pad_dst])
    src_p = src_p.reshape(IDX_ROWS, LANES)
    dst_p = dst_p.reshape(IDX_ROWS, LANES)

    s1, p1 = _proj1(x, W_self1, W_neigh1)
    aggp1 = _sc_agg_w1(p1, src_p, dst_p)

    s2, p2, deg = _layer2(s1, aggp1, b1.reshape(1, H), W_self2, W_neigh2)

    aggp2 = _sc_agg_w2(p2, src_p, dst_p)

    return _head(s2, aggp2, deg, b2.reshape(1, H), W_head,
                 b_head.reshape(1, O))


# speedup vs baseline: 3.0980x; 1.0330x over previous
import functools

import jax
import jax.numpy as jnp
from jax import lax
from jax.experimental import pallas as pl
from jax.experimental.pallas import tpu as pltpu
from jax.experimental.pallas import tpu_sc as plsc

N = 10000
E = 320000
D = 128
H = 64
O = 2
W1 = H + 16

NC = 2
NS = 16
NW = NC * NS

LANES = 128
CH = 10
HB = CH // 2
R = 10240
ROWS_PER_TILE = R // NS
E_PAD = 327680
IDX_ROWS = E_PAD // LANES
ROWS_PER_WORKER = IDX_ROWS // NW

BN = 1000
GRID = N // BN


def _make_sc_aggregate(width):
    mesh = plsc.VectorSubcoreMesh(core_axis_name="c", subcore_axis_name="s",
                                  num_cores=NC, num_subcores=NS)
    t_rows = N // NS

    def body(p_hbm, src_hbm, dst_hbm, agg_out,
             srcc_v, dstc_v, rows_v, table_sh, acc_sh, sem_g, sem_s, sem_i):
        cid = lax.axis_index("c")
        sid = lax.axis_index("s")
        wid = cid * NS + sid
        r0 = sid * ROWS_PER_TILE
        base = wid * ROWS_PER_WORKER

        ct = pltpu.async_copy(p_hbm.at[pl.ds(sid * t_rows, t_rows)],
                              table_sh.at[pl.ds(sid * t_rows, t_rows)],
                              sem_i)

        def zstore(i, carry):
            for k in range(width // 16):
                rows_v[0, pl.ds(i * 2, 2), pl.ds(k * 16, 16)] = jnp.zeros(
                    (2, 16), jnp.bfloat16)
            return carry

        lax.fori_loop(0, LANES // 2, zstore, 0)
        for q in range(ROWS_PER_TILE // LANES):
            pltpu.sync_copy(rows_v.at[0],
                            acc_sh.at[pl.ds(r0 + q * LANES, LANES)])
        ct.wait()
        plsc.subcore_barrier()

        def chunk(c, carry):
            crow = base + c * CH
            ci0 = pltpu.async_copy(src_hbm.at[pl.ds(crow, CH)], srcc_v,
                                   sem_i)
            ci1 = pltpu.async_copy(dst_hbm.at[pl.ds(crow, CH)], dstc_v,
                                   sem_i)
            ci0.wait()
            ci1.wait()

            g0 = [pltpu.async_copy(table_sh.at[srcc_v.at[j]],
                                   rows_v.at[j], sem_g)
                  for j in range(HB)]
            scatters = []
            g1 = []
            for j in range(HB):
                g0[j].wait()
                scatters.append(
                    pltpu.async_copy(rows_v.at[j],
                                     acc_sh.at[dstc_v.at[j]],
                                     sem_s, add=True))
                g1.append(
                    pltpu.async_copy(table_sh.at[srcc_v.at[HB + j]],
                                     rows_v.at[HB + j], sem_g))
            for j in range(HB):
                g1[j].wait()
                scatters.append(
                    pltpu.async_copy(rows_v.at[HB + j],
                                     acc_sh.at[dstc_v.at[HB + j]],
                                     sem_s, add=True))
            for h in scatters:
                h.wait()
            return carry

        lax.fori_loop(0, ROWS_PER_WORKER // CH, chunk, 0)
        plsc.subcore_barrier()

        pltpu.sync_copy(acc_sh.at[pl.ds(r0, ROWS_PER_TILE)],
                        agg_out.at[cid, pl.ds(r0, ROWS_PER_TILE)])

    return pl.kernel(
        body,
        out_type=jax.ShapeDtypeStruct((NC, R, width), jnp.bfloat16),
        mesh=mesh,
        scratch_types=(
            pltpu.VMEM((CH, LANES), jnp.int32),
            pltpu.VMEM((CH, LANES), jnp.int32),
            pltpu.VMEM((CH, LANES, width), jnp.bfloat16),
            pltpu.VMEM_SHARED((N, width), jnp.bfloat16),
            pltpu.VMEM_SHARED((R, width), jnp.bfloat16),
            pltpu.SemaphoreType.DMA,
            pltpu.SemaphoreType.DMA,
            pltpu.SemaphoreType.DMA,
        ),
        compiler_params=pltpu.CompilerParams(use_tc_tiling_on_sc=False))


_sc_agg_w1 = _make_sc_aggregate(W1)
_sc_agg_w2 = _make_sc_aggregate(H)


def _proj1_body(x_ref, wa_ref, wb_ref, oa_ref, ob_ref):
    xb = x_ref[...]
    oa_ref[...] = jnp.dot(xb, wa_ref[...], preferred_element_type=jnp.float32)
    pb = jnp.dot(xb, wb_ref[...], preferred_element_type=jnp.float32)
    ob_ref[...] = jnp.concatenate(
        [pb, jnp.ones((BN, W1 - H), jnp.float32)], axis=1).astype(jnp.bfloat16)


def _proj1(x, wa, wb):
    return pl.pallas_call(
        _proj1_body,
        grid=(GRID,),
        in_specs=[
            pl.BlockSpec((BN, D), lambda i: (i, 0)),
            pl.BlockSpec((D, H), lambda i: (0, 0)),
            pl.BlockSpec((D, H), lambda i: (0, 0)),
        ],
        out_specs=[
            pl.BlockSpec((BN, H), lambda i: (i, 0)),
            pl.BlockSpec((BN, W1), lambda i: (i, 0)),
        ],
        out_shape=[
            jax.ShapeDtypeStruct((N, H), jnp.float32),
            jax.ShapeDtypeStruct((N, W1), jnp.bfloat16),
        ],
    )(x, wa, wb)


def _layer2_body(s_ref, aggp_ref, b_ref, wa_ref, wb_ref,
                 oa_ref, ob_ref, od_ref):
    comb = (aggp_ref[0].astype(jnp.float32)
            + aggp_ref[1].astype(jnp.float32))
    deg = comb[:, H:H + 1]
    h = jnp.maximum(
        s_ref[...] + comb[:, :H] / jnp.maximum(deg, 1.0) + b_ref[...], 0.0)
    oa_ref[...] = jnp.dot(h, wa_ref[...], preferred_element_type=jnp.float32)
    ob_ref[...] = jnp.dot(h, wb_ref[...],
                          preferred_element_type=jnp.float32
                          ).astype(jnp.bfloat16)
    od_ref[...] = comb[:, H:]


def _layer2(s, aggp, b, wa, wb):
    return pl.pallas_call(
        _layer2_body,
        grid=(GRID,),
        in_specs=[
            pl.BlockSpec((BN, H), lambda i: (i, 0)),
            pl.BlockSpec((NC, BN, W1), lambda i: (0, i, 0)),
            pl.BlockSpec((1, H), lambda i: (0, 0)),
            pl.BlockSpec((H, H), lambda i: (0, 0)),
            pl.BlockSpec((H, H), lambda i: (0, 0)),
        ],
        out_specs=[
            pl.BlockSpec((BN, H), lambda i: (i, 0)),
            pl.BlockSpec((BN, H), lambda i: (i, 0)),
            pl.BlockSpec((BN, W1 - H), lambda i: (i, 0)),
        ],
        out_shape=[
            jax.ShapeDtypeStruct((N, H), jnp.float32),
            jax.ShapeDtypeStruct((N, H), jnp.bfloat16),
            jax.ShapeDtypeStruct((N, W1 - H), jnp.float32),
        ],
    )(s, aggp, b, wa, wb)


def _head_body(s_ref, aggp_ref, deg_ref, b_ref, wh_ref, bh_ref, o_ref):
    agg = (aggp_ref[0].astype(jnp.float32)
           + aggp_ref[1].astype(jnp.float32))
    deg = deg_ref[:, 0:1]
    h = jnp.maximum(
        s_ref[...] + agg / jnp.maximum(deg, 1.0) + b_ref[...], 0.0)
    o_ref[...] = (jnp.dot(h, wh_ref[...], preferred_element_type=jnp.float32)
                  + bh_ref[...])


def _head(s, aggp, deg, b, wh, bh):
    return pl.pallas_call(
        _head_body,
        grid=(GRID,),
        in_specs=[
            pl.BlockSpec((BN, H), lambda i: (i, 0)),
            pl.BlockSpec((NC, BN, H), lambda i: (0, i, 0)),
            pl.BlockSpec((BN, W1 - H), lambda i: (i, 0)),
            pl.BlockSpec((1, H), lambda i: (0, 0)),
            pl.BlockSpec((H, O), lambda i: (0, 0)),
            pl.BlockSpec((1, O), lambda i: (0, 0)),
        ],
        out_specs=pl.BlockSpec((BN, O), lambda i: (i, 0)),
        out_shape=jax.ShapeDtypeStruct((N, O), jnp.float32),
    )(s, aggp, deg, b, wh, bh)


def kernel(x, edge_index, W_self1, W_neigh1, b1, W_self2, W_neigh2, b2,
           W_head, b_head):
    src = edge_index[0].astype(jnp.int32)
    dst = edge_index[1].astype(jnp.int32)
    pad = E_PAD - E
    src_p = jnp.concatenate([src, jnp.zeros((pad,), jnp.int32)])
    dst_p = jnp.concatenate([dst, jnp.full((pad,), N, jnp.int32)])
    src_p = src_p.reshape(IDX_ROWS, LANES)
    dst_p = dst_p.reshape(IDX_ROWS, LANES)

    s1, p1 = _proj1(x, W_self1, W_neigh1)
    aggp1 = _sc_agg_w1(p1, src_p, dst_p)

    s2, p2, deg = _layer2(s1, aggp1, b1.reshape(1, H), W_self2, W_neigh2)

    aggp2 = _sc_agg_w2(p2, src_p, dst_p)

    return _head(s2, aggp2, deg, b2.reshape(1, H), W_head,
                 b_head.reshape(1, O))
